# baseline (speedup 1.0000x reference)
import functools

import jax
import jax.numpy as jnp
from jax import lax
from jax.experimental import pallas as pl
from jax.experimental.pallas import tpu as pltpu
from jax.experimental.pallas import tpu_sc as plsc

N = 10000
E = 320000
D = 128
NC = 2
NS = 16
NW = NC * NS
EPW = E // NW
B = 128
CR = 80
CS = 88
CP = 96
NP = 10240
RPT = NP // NS

_mesh = plsc.VectorSubcoreMesh(core_axis_name="c", subcore_axis_name="s")


@functools.partial(
    pl.kernel,
    mesh=_mesh,
    out_type=jax.ShapeDtypeStruct((NW, 1, N), jnp.float32),
    scratch_types=[
        pltpu.VMEM((1, EPW), jnp.int32),
        pltpu.VMEM((N,), jnp.float32),
    ],
    compiler_params=pltpu.CompilerParams(needs_layout_passes=False),
)
def _p1_degree(row_hbm, degp_hbm, idx_v, deg_v):
    c = lax.axis_index("c")
    s = lax.axis_index("s")
    wid = c * NS + s

    pltpu.sync_copy(row_hbm.at[wid], idx_v)

    def zero_body(i, _):
        deg_v[pl.ds(i * 16, 16)] = jnp.zeros((16,), jnp.float32)
        return 0

    lax.fori_loop(0, N // 16, zero_body, 0)

    ones16 = jnp.ones((16,), jnp.float32)

    def hist_body(j, _):
        idx = idx_v[0, pl.ds(j * 16, 16)]
        plsc.addupdate_scatter(deg_v, [idx], ones16)
        return 0

    lax.fori_loop(0, EPW // 16, hist_body, 0)
    pltpu.sync_copy(deg_v, degp_hbm.at[wid, 0])


@functools.partial(
    pl.kernel,
    mesh=_mesh,
    out_type=jax.ShapeDtypeStruct((NC, NP, D), jnp.float32),
    scratch_types=[
        pltpu.VMEM((CS, B), jnp.int32),
        pltpu.VMEM((2, 8, B), jnp.int32),
        pltpu.VMEM((B, D), jnp.float32),
        pltpu.VMEM((B, D), jnp.float32),
        pltpu.VMEM((16, D), jnp.float32),
        pltpu.VMEM_SHARED((NP, D), jnp.float32),
        pltpu.SemaphoreType.DMA,
        pltpu.SemaphoreType.DMA,
        pltpu.SemaphoreType.DMA,
        pltpu.SemaphoreType.DMA,
    ],
)
def _p3_scatter(y_hbm, col_hbm, row_hbm, zp_hbm, col_v, rowr, buf0, buf1,
                zb_v, z_sh, semg0, semg1, semr0, semr1):
    c = lax.axis_index("c")
    s = lax.axis_index("s")
    wid = c * NS + s

    pltpu.sync_copy(col_hbm.at[wid, pl.ds(0, CS)], col_v)
    for i in range(16):
        for k in range(D // 16):
            zb_v[i, pl.ds(k * 16, 16)] = jnp.zeros((16,), jnp.float32)

    def zero_body(i, _):
        pltpu.async_copy(zb_v, z_sh.at[pl.ds(s * RPT + i * 16, 16)], semg0)
        return 0

    lax.fori_loop(0, RPT // 16, zero_body, 0)

    def zero_drain(i, _):
        pltpu.make_async_copy(zb_v, z_sh.at[pl.ds(s * RPT, 16)], semg0).wait()
        return 0

    lax.fori_loop(0, RPT // 16, zero_drain, 0)
    plsc.subcore_barrier()

    bufs = (buf0, buf1)
    semg = (semg0, semg1)

    def row_fetch(chunk0, slot, sem):
        return pltpu.async_copy(
            row_hbm.at[wid, pl.ds(pl.multiple_of(chunk0, 8), 8)],
            rowr.at[slot], sem)

    def row_wait(slot, sem):
        pltpu.make_async_copy(row_hbm.at[wid, pl.ds(0, 8)],
                              rowr.at[slot], sem).wait()

    def gather(chunk, m):
        return pltpu.async_copy(y_hbm.at[col_v.at[chunk]], bufs[m], semg[m])

    def gather_wait(m):
        pltpu.make_async_copy(y_hbm.at[col_v.at[0]], bufs[m], semg[m]).wait()

    row_fetch(0, 0, semr0)
    row_fetch(8, 1, semr1)
    gather(0, 0)
    gather(1, 1)

    def block_pair(i, _):
        base = 16 * i
        row_wait(0, semr0)
        for tt in range(8):
            m = tt % 2
            gather_wait(m)
            pltpu.sync_copy(bufs[m], z_sh.at[rowr.at[0, tt]], add=True)
            gather(base + tt + 2, m)
        row_fetch(base + 16, 0, semr0)
        row_wait(1, semr1)
        for tt in range(8):
            m = tt % 2
            gather_wait(m)
            pltpu.sync_copy(bufs[m], z_sh.at[rowr.at[1, tt]], add=True)
            gather(base + 8 + tt + 2, m)
        row_fetch(base + 24, 1, semr1)
        return 0

    lax.fori_loop(0, CR // 16, block_pair, 0)
    row_wait(0, semr0)
    row_wait(1, semr1)
    gather_wait(0)
    gather_wait(1)
    plsc.subcore_barrier()

    pltpu.sync_copy(z_sh.at[pl.ds(s * RPT, RPT)],
                    zp_hbm.at[c, pl.ds(s * RPT, RPT)])


_R = 400
_G = N // _R


def _deg_inv_block(degp_blk):
    d = jnp.sum(degp_blk, axis=0).reshape(_R, 1)
    safe = jnp.where(d > 0, d, 1.0)
    return jnp.where(d > 0, lax.rsqrt(safe), 0.0)


def _p2_body(x_ref, degp_ref, w_ref, y_ref):
    dinv = _deg_inv_block(degp_ref[:])
    y_ref[:] = jnp.dot(x_ref[:], w_ref[:],
                       preferred_element_type=jnp.float32) * dinv


def _p2_scale_matmul(x, degp4, W):
    return pl.pallas_call(
        _p2_body,
        grid=(_G,),
        in_specs=[
            pl.BlockSpec((_R, D), lambda i: (i, 0)),
            pl.BlockSpec((NW, 1, 1, _R), lambda i: (0, i, 0, 0)),
            pl.BlockSpec((D, D), lambda i: (0, 0)),
        ],
        out_specs=pl.BlockSpec((_R, D), lambda i: (i, 0)),
        out_shape=jax.ShapeDtypeStruct((N, D), jnp.float32),
    )(x, degp4, W)


def _p4_body(zp_ref, degp_ref, out_ref):
    dinv = _deg_inv_block(degp_ref[:])
    out_ref[:] = (zp_ref[0] + zp_ref[1]) * dinv


def _p4_combine(zp, degp4):
    return pl.pallas_call(
        _p4_body,
        grid=(_G,),
        in_specs=[
            pl.BlockSpec((NC, _R, D), lambda i: (0, i, 0)),
            pl.BlockSpec((NW, 1, 1, _R), lambda i: (0, i, 0, 0)),
        ],
        out_specs=pl.BlockSpec((_R, D), lambda i: (i, 0)),
        out_shape=jax.ShapeDtypeStruct((N, D), jnp.float32),
    )(zp, degp4)


def kernel(x, edge_index, edge_type, r, W):
    del edge_type, r
    ei = edge_index.astype(jnp.int32)
    pad = CP * B - EPW
    row3 = jnp.pad(ei[0].reshape(NW, EPW), ((0, 0), (0, pad)),
                   constant_values=N + 200).reshape(NW, CP, B)
    col3 = jnp.pad(ei[1].reshape(NW, EPW), ((0, 0), (0, pad)),
                   constant_values=0).reshape(NW, CP, B)
    degp = _p1_degree(ei[0].reshape(NW, 1, EPW))
    degp4 = degp.reshape(NW, _G, 1, _R)
    y = _p2_scale_matmul(x, degp4, W)
    zp = _p3_scatter(y, col3, row3)
    return _p4_combine(zp, degp4)


# speedup vs baseline: 3.6912x; 3.6912x over previous
import functools

import jax
import jax.numpy as jnp
from jax import lax
from jax.experimental import pallas as pl
from jax.experimental.pallas import tpu as pltpu
from jax.experimental.pallas import tpu_sc as plsc

N = 10000
E = 320000
D = 128
NC = 2
NS = 16
NW = NC * NS
EPW = E // NW
B = 125
C = EPW // B
NP = 10240
RPT = NP // NS

_mesh = plsc.VectorSubcoreMesh(core_axis_name="c", subcore_axis_name="s")


@functools.partial(
    pl.kernel,
    mesh=_mesh,
    out_type=jax.ShapeDtypeStruct((NW, 1, N), jnp.float32),
    scratch_types=[
        pltpu.VMEM((1, EPW), jnp.int32),
        pltpu.VMEM((N,), jnp.float32),
    ],
    compiler_params=pltpu.CompilerParams(needs_layout_passes=False),
)
def _p1_degree(row_hbm, degp_hbm, idx_v, deg_v):
    c = lax.axis_index("c")
    s = lax.axis_index("s")
    wid = c * NS + s

    pltpu.sync_copy(row_hbm.at[wid], idx_v)

    def zero_body(i, _):
        deg_v[pl.ds(i * 16, 16)] = jnp.zeros((16,), jnp.float32)
        return 0

    lax.fori_loop(0, N // 16, zero_body, 0)

    ones16 = jnp.ones((16,), jnp.float32)

    def hist_body(j, _):
        idx = idx_v[0, pl.ds(j * 16, 16)]
        plsc.addupdate_scatter(deg_v, [idx], ones16)
        return 0

    lax.fori_loop(0, EPW // 16, hist_body, 0)
    pltpu.sync_copy(deg_v, degp_hbm.at[wid, 0])


@functools.partial(
    pl.kernel,
    mesh=_mesh,
    out_type=jax.ShapeDtypeStruct((NC, NP, D), jnp.float32),
    scratch_types=[
        pltpu.VMEM((16, B), jnp.int32),
        pltpu.VMEM((16, B), jnp.int32),
        pltpu.VMEM((B, D), jnp.float32),
        pltpu.VMEM((B, D), jnp.float32),
        pltpu.VMEM((16, D), jnp.float32),
        pltpu.VMEM_SHARED((NP, D), jnp.float32),
        pltpu.SemaphoreType.DMA,
        pltpu.SemaphoreType.DMA,
        pltpu.SemaphoreType.DMA,
        pltpu.SemaphoreType.DMA,
        pltpu.SemaphoreType.DMA,
        pltpu.SemaphoreType.DMA,
    ],
)
def _p3_scatter(y_hbm, col_hbm, row_hbm, zp_hbm, colr, rowr, buf0, buf1,
                zb_v, z_sh, semg0, semg1, sems0, sems1, semc, semr):
    c = lax.axis_index("c")
    s = lax.axis_index("s")
    wid = c * NS + s

    for i in range(16):
        for k in range(D // 16):
            zb_v[i, pl.ds(k * 16, 16)] = jnp.zeros((16,), jnp.float32)

    def zero_body(i, _):
        pltpu.sync_copy(zb_v, z_sh.at[pl.ds(s * RPT + i * 16, 16)])
        return 0

    lax.fori_loop(0, RPT // 16, zero_body, 0)
    plsc.subcore_barrier()

    bufs = (buf0, buf1)
    semg = (semg0, semg1)
    sems = (sems0, sems1)

    def g_issue(t, m):
        pltpu.async_copy(y_hbm.at[colr.at[t % 16]], bufs[m], semg[m])

    def g_wait(m):
        pltpu.make_async_copy(y_hbm.at[colr.at[0]], bufs[m], semg[m]).wait()

    def s_issue(k, m):
        pltpu.async_copy(bufs[m], z_sh.at[rowr.at[k]], sems[m], add=True)

    def s_wait(m):
        pltpu.make_async_copy(bufs[m], z_sh.at[rowr.at[0]], sems[m]).wait()

    pltpu.sync_copy(col_hbm.at[wid, pl.ds(0, 16)], colr)
    pltpu.sync_copy(row_hbm.at[wid, pl.ds(0, 16)], rowr)
    pltpu.async_copy(y_hbm.at[colr.at[0]], buf0, semg0)

    def block_body(i, _):
        base = 16 * i

        def fetch(hbm, ring, slot, chunk0, sem):
            pltpu.async_copy(
                hbm.at[wid, pl.ds(pl.multiple_of(chunk0, 8), 8)],
                ring.at[pl.ds(slot * 8, 8)], sem)

        def fwait(hbm, ring, slot, sem):
            pltpu.make_async_copy(hbm.at[wid, pl.ds(0, 8)],
                                  ring.at[pl.ds(slot * 8, 8)], sem).wait()

        for kk in range(16):
            m = kk % 2
            g_wait(m)
            if kk == 0:
                @pl.when(i > 0)
                def _():
                    s_wait(1 - m)
                    fetch(col_hbm, colr, 1, base + 8, semc)
                    fetch(row_hbm, rowr, 1, base + 8, semr)
            else:
                s_wait(1 - m)
            if kk == 7:
                @pl.when(i > 0)
                def _():
                    fwait(col_hbm, colr, 1, semc)
            if kk == 8:
                @pl.when(i > 0)
                def _():
                    fwait(row_hbm, rowr, 1, semr)
                @pl.when(i < 4)
                def _():
                    fetch(col_hbm, colr, 0, base + 16, semc)
            if kk == 9:
                @pl.when(i < 4)
                def _():
                    fetch(row_hbm, rowr, 0, base + 16, semr)
            if kk == 14:
                @pl.when(i < 4)
                def _():
                    fwait(col_hbm, colr, 0, semc)
                    fwait(row_hbm, rowr, 0, semr)
            if kk == 15:
                @pl.when(i < 4)
                def _():
                    g_issue(base + kk + 1, 1 - m)
            else:
                g_issue(base + kk + 1, 1 - m)
            s_issue(kk, m)
        return 0

    lax.fori_loop(0, C // 16, block_body, 0)
    s_wait(1)
    plsc.subcore_barrier()

    pltpu.sync_copy(z_sh.at[pl.ds(s * RPT, RPT)],
                    zp_hbm.at[c, pl.ds(s * RPT, RPT)])


_R = 400
_G = N // _R


def _deg_inv_block(degp_blk):
    d = jnp.sum(degp_blk, axis=0).reshape(_R, 1)
    safe = jnp.where(d > 0, d, 1.0)
    return jnp.where(d > 0, lax.rsqrt(safe), 0.0)


def _p2_body(x_ref, degp_ref, w_ref, y_ref):
    dinv = _deg_inv_block(degp_ref[:])
    y_ref[:] = jnp.dot(x_ref[:], w_ref[:],
                       preferred_element_type=jnp.float32) * dinv


def _p2_scale_matmul(x, degp4, W):
    return pl.pallas_call(
        _p2_body,
        grid=(_G,),
        in_specs=[
            pl.BlockSpec((_R, D), lambda i: (i, 0)),
            pl.BlockSpec((NW, 1, 1, _R), lambda i: (0, i, 0, 0)),
            pl.BlockSpec((D, D), lambda i: (0, 0)),
        ],
        out_specs=pl.BlockSpec((_R, D), lambda i: (i, 0)),
        out_shape=jax.ShapeDtypeStruct((N, D), jnp.float32),
    )(x, degp4, W)


def _p4_body(zp_ref, degp_ref, out_ref):
    dinv = _deg_inv_block(degp_ref[:])
    out_ref[:] = (zp_ref[0] + zp_ref[1]) * dinv


def _p4_combine(zp, degp4):
    return pl.pallas_call(
        _p4_body,
        grid=(_G,),
        in_specs=[
            pl.BlockSpec((NC, _R, D), lambda i: (0, i, 0)),
            pl.BlockSpec((NW, 1, 1, _R), lambda i: (0, i, 0, 0)),
        ],
        out_specs=pl.BlockSpec((_R, D), lambda i: (i, 0)),
        out_shape=jax.ShapeDtypeStruct((N, D), jnp.float32),
    )(zp, degp4)


def kernel(x, edge_index, edge_type, r, W):
    del edge_type, r
    ei = edge_index.astype(jnp.int32)
    row3 = ei[0].reshape(NW, C, B)
    col3 = ei[1].reshape(NW, C, B)
    degp = _p1_degree(ei[0].reshape(NW, 1, EPW))
    degp4 = degp.reshape(NW, _G, 1, _R)
    y = _p2_scale_matmul(x, degp4, W)
    zp = _p3_scatter(y, col3, row3)
    return _p4_combine(zp, degp4)
